# baseline (speedup 1.0000x reference)
import functools

import jax
import jax.numpy as jnp
from jax import lax
from jax.experimental import pallas as pl
from jax.experimental.pallas import tpu as pltpu
from jax.experimental.pallas import tpu_sc as plsc

_NUM_CORES = 2
_NUM_SUBCORES = 16
_NUM_WORKERS = _NUM_CORES * _NUM_SUBCORES
_LANES = 16
_IDX = 128
_BLK = 256


def _sc_broadcast(rows, seq_len, dim, embeddings):
    full_steps = rows // _NUM_WORKERS
    rem_rows = rows - full_steps * _NUM_WORKERS
    writes_per_row = seq_len // _BLK
    gathers_per_blk = _BLK // _IDX
    rem_chunk = seq_len // _NUM_WORKERS

    mesh = plsc.VectorSubcoreMesh(core_axis_name="c", subcore_axis_name="s")

    @functools.partial(
        pl.kernel,
        out_type=jax.ShapeDtypeStruct((rows, seq_len, dim), jnp.float32),
        mesh=mesh,
        scratch_types=[
            pltpu.VMEM((2, _IDX), jnp.int32),
            pltpu.VMEM((2, _BLK, dim), jnp.float32),
            pltpu.SemaphoreType.DMA((2,)),
            pltpu.SemaphoreType.DMA((2,)),
        ],
    )
    def kern(emb_hbm, out_hbm, idx_v, buf_v, gsem, wsem):
        wid = lax.axis_index("s") * _NUM_CORES + lax.axis_index("c")

        def fire_gather(slot, row):
            for v in range(_IDX // _LANES):
                idx_v[slot, pl.ds(v * _LANES, _LANES)] = jnp.full(
                    (_LANES,), row, jnp.int32
                )
            return [
                pltpu.async_copy(
                    emb_hbm.at[idx_v.at[slot]],
                    buf_v.at[slot, pl.ds(k * _IDX, _IDX), :],
                    gsem.at[slot],
                )
                for k in range(gathers_per_blk)
            ]

        def fire_writes(slot, row):
            return [
                pltpu.async_copy(
                    buf_v.at[slot],
                    out_hbm.at[row, pl.ds(j * _BLK, _BLK), :],
                    wsem.at[slot],
                )
                for j in range(writes_per_row)
            ]

        pend_g = {0: [], 1: []}
        pend_w = {0: [], 1: []}

        if full_steps:
            pend_g[0] = fire_gather(0, wid)
        for step in range(full_steps):
            slot = step % 2
            row = step * _NUM_WORKERS + wid
            for cp in pend_g[slot]:
                cp.wait()
            pend_w[slot] = fire_writes(slot, row)
            if step + 1 < full_steps:
                for cp in pend_w[1 - slot]:
                    cp.wait()
                pend_w[1 - slot] = []
                pend_g[1 - slot] = fire_gather(
                    1 - slot, (step + 1) * _NUM_WORKERS + wid
                )

        for r in range(rem_rows):
            row = full_steps * _NUM_WORKERS + r
            slot = (full_steps + r) % 2
            for cp in pend_w[slot]:
                cp.wait()
            pend_w[slot] = []
            pend_g[slot] = fire_gather(slot, row)
            for cp in pend_g[slot]:
                cp.wait()
            pend_w[slot] = [
                pltpu.async_copy(
                    buf_v.at[slot, pl.ds(0, rem_chunk), :],
                    out_hbm.at[row, pl.ds(wid * rem_chunk, rem_chunk), :],
                    wsem.at[slot],
                )
            ]

        for slot in (0, 1):
            for cp in pend_w[slot]:
                cp.wait()

    return kern(embeddings)


def kernel(time, embeddings):
    batch_size, seq_len = time.shape
    rows, dim = embeddings.shape
    return _sc_broadcast(rows, seq_len, dim, embeddings)


# speedup vs baseline: 1.6413x; 1.6413x over previous
import functools

import jax
import jax.numpy as jnp
from jax import lax
from jax.experimental import pallas as pl
from jax.experimental.pallas import tpu as pltpu
from jax.experimental.pallas import tpu_sc as plsc

_NUM_CORES = 2
_NUM_SUBCORES = 16
_NUM_WORKERS = _NUM_CORES * _NUM_SUBCORES
_LANES = 16
_BLK = 128


def _sc_broadcast(rows, seq_len, dim, embeddings):
    full_steps = rows // _NUM_WORKERS
    rem_rows = rows - full_steps * _NUM_WORKERS
    writes_per_row = seq_len // _BLK
    rem_chunk = seq_len // _NUM_WORKERS

    mesh = plsc.VectorSubcoreMesh(core_axis_name="c", subcore_axis_name="s")

    @functools.partial(
        pl.kernel,
        out_type=jax.ShapeDtypeStruct((rows, seq_len, dim), jnp.float32),
        mesh=mesh,
        scratch_types=[
            pltpu.VMEM((_BLK,), jnp.int32),
            pltpu.VMEM((_BLK,), jnp.int32),
            pltpu.VMEM((_BLK, dim), jnp.float32),
            pltpu.VMEM((_BLK, dim), jnp.float32),
            pltpu.SemaphoreType.DMA,
            pltpu.SemaphoreType.DMA,
            pltpu.SemaphoreType.DMA,
            pltpu.SemaphoreType.DMA,
        ],
    )
    def kern(emb_hbm, out_hbm, idx0, idx1, buf0, buf1, gs0, gs1, ws0, ws1):
        wid = lax.axis_index("s") * _NUM_CORES + lax.axis_index("c")
        idx = (idx0, idx1)
        buf = (buf0, buf1)
        gsem = (gs0, gs1)
        wsem = (ws0, ws1)

        def fire_gather(slot, row):
            for v in range(_BLK // _LANES):
                idx[slot][pl.ds(v * _LANES, _LANES)] = jnp.full(
                    (_LANES,), row, jnp.int32
                )
            return [
                pltpu.async_copy(
                    emb_hbm.at[idx[slot]], buf[slot], gsem[slot]
                )
            ]

        def fire_writes(slot, row):
            return [
                pltpu.async_copy(
                    buf[slot],
                    out_hbm.at[row, pl.ds(j * _BLK, _BLK), :],
                    wsem[slot],
                )
                for j in range(writes_per_row)
            ]

        pend_g = {0: [], 1: []}
        pend_w = {0: [], 1: []}

        if full_steps:
            pend_g[0] = fire_gather(0, wid)
        for step in range(full_steps):
            slot = step % 2
            row = step * _NUM_WORKERS + wid
            for cp in pend_g[slot]:
                cp.wait()
            pend_w[slot] = fire_writes(slot, row)
            if step + 1 < full_steps:
                for cp in pend_w[1 - slot]:
                    cp.wait()
                pend_w[1 - slot] = []
                pend_g[1 - slot] = fire_gather(
                    1 - slot, (step + 1) * _NUM_WORKERS + wid
                )

        for r in range(rem_rows):
            row = full_steps * _NUM_WORKERS + r
            slot = (full_steps + r) % 2
            for cp in pend_w[slot]:
                cp.wait()
            pend_w[slot] = []
            pend_g[slot] = fire_gather(slot, row)
            for cp in pend_g[slot]:
                cp.wait()
            pend_w[slot] = [
                pltpu.async_copy(
                    buf[slot].at[pl.ds(0, rem_chunk), :],
                    out_hbm.at[row, pl.ds(wid * rem_chunk, rem_chunk), :],
                    wsem[slot],
                )
            ]

        for slot in (0, 1):
            for cp in pend_w[slot]:
                cp.wait()

    return kern(embeddings)


def kernel(time, embeddings):
    batch_size, seq_len = time.shape
    rows, dim = embeddings.shape
    return _sc_broadcast(rows, seq_len, dim, embeddings)
